# baseline (device time: 8321 ns/iter reference)
import jax
import jax.numpy as jnp
from jax import lax
from jax.experimental import pallas as pl
from jax.experimental.pallas import tpu as pltpu

K = 8
CHUNKS = 8

_NET8 = [
    (0, 1), (2, 3), (4, 5), (6, 7),
    (0, 2), (1, 3), (4, 6), (5, 7),
    (1, 2), (5, 6),
    (0, 4), (1, 5), (2, 6), (3, 7),
    (2, 4), (3, 5),
    (1, 2), (3, 4), (5, 6),
]


def _topk_rows_chunked(xs, k):
    m, n = xs.shape
    w = n // CHUNKS
    neg = jnp.finfo(xs.dtype).min
    s = [xs[:, i * w : (i + 1) * w] for i in range(CHUNKS)]
    for i, j in _NET8:
        hi = jnp.maximum(s[i], s[j])
        lo = jnp.minimum(s[i], s[j])
        s[i], s[j] = hi, lo
    out = []
    for _ in range(k):
        mx = jnp.max(s[0], axis=1, keepdims=True)
        out.append(mx)
        mask = s[0] == mx
        for j in range(CHUNKS - 1):
            s[j] = jnp.where(mask, s[j + 1], s[j])
        s[CHUNKS - 1] = jnp.where(mask, neg, s[CHUNKS - 1])
    return jnp.concatenate(out, axis=1)


def _topk_rows_small(vals, k):
    neg = jnp.finfo(vals.dtype).min
    out = []
    cur = vals
    for _ in range(k):
        mx = jnp.max(cur, axis=1, keepdims=True)
        out.append(mx)
        cur = jnp.where(cur == mx, neg, cur)
    return jnp.concatenate(out, axis=1)


def kernel(x):
    m, n = x.shape

    def body(x_ref, out_ref, local_ref, recv_ref, send_sem, recv_sem):
        my_x = lax.axis_index("x")
        my_y = lax.axis_index("y")
        my_z = lax.axis_index("z")
        partner = (1 - my_x, my_y, my_z)

        barrier_sem = pltpu.get_barrier_semaphore()
        pl.semaphore_signal(
            barrier_sem,
            inc=1,
            device_id=partner,
            device_id_type=pl.DeviceIdType.MESH,
        )

        local_ref[:, :] = _topk_rows_chunked(x_ref[:, :], K)

        pl.semaphore_wait(barrier_sem, 1)

        rdma = pltpu.make_async_remote_copy(
            src_ref=local_ref,
            dst_ref=recv_ref,
            send_sem=send_sem,
            recv_sem=recv_sem,
            device_id=partner,
            device_id_type=pl.DeviceIdType.MESH,
        )
        rdma.start()
        rdma.wait()

        merged = jnp.concatenate([local_ref[:, :], recv_ref[:, :]], axis=1)
        out_ref[:, :] = _topk_rows_small(merged, K).astype(jnp.float32)

    return pl.pallas_call(
        body,
        out_shape=jax.ShapeDtypeStruct((m, K), jnp.float32),
        in_specs=[pl.BlockSpec(memory_space=pltpu.VMEM)],
        out_specs=pl.BlockSpec(memory_space=pltpu.VMEM),
        scratch_shapes=[
            pltpu.VMEM((m, K), x.dtype),
            pltpu.VMEM((m, K), x.dtype),
            pltpu.SemaphoreType.DMA,
            pltpu.SemaphoreType.DMA,
        ],
        compiler_params=pltpu.CompilerParams(collective_id=0),
    )(x)
